# Initial kernel scaffold; baseline (speedup 1.0000x reference)
import jax
import jax.numpy as jnp
from jax.experimental import pallas as pl


def kernel(query, reference_points, value, W_value, b_value, W_off, b_off, W_att, b_att, W_out, b_out, value_spatial_shapes, value_level_start_index):
    raise NotImplementedError("write your pallas kernel here")


# speedup vs baseline: 2027.6276x; 2027.6276x over previous
import functools

import jax
import jax.numpy as jnp
from jax import lax
from jax.experimental import pallas as pl
from jax.experimental.pallas import tpu as pltpu
from jax.experimental.pallas import tpu_sc as plsc

HEADS = 8
LEVELS = 4
POINTS = 4
EMBED = 256
HEAD_DIM = 32
BS = 2
NQ = 5440
NV = 5440
NQB = BS * NQ
ENTRIES = 4 * HEADS * LEVELS * POINTS
QBLK = 128
NBLK = NQB // QBLK

_SIZES = (64, 32, 16, 8)
_STARTS = (0, 4096, 5120, 5376)

_ATT_PERM = tuple(h * 16 + p * 4 + l
                  for h in range(HEADS)
                  for l in range(LEVELS)
                  for p in range(POINTS))


def _matmul_body(x_ref, w_ref, b_ref, o_ref):
    o_ref[...] = jnp.dot(x_ref[...], w_ref[...],
                         preferred_element_type=jnp.float32) + b_ref[...]


def _tc_matmul(x_bf, w_bf, b):
    m, k = x_bf.shape
    n = w_bf.shape[1]
    return pl.pallas_call(
        _matmul_body,
        grid=(m // QBLK,),
        in_specs=[
            pl.BlockSpec((QBLK, k), lambda i: (i, 0)),
            pl.BlockSpec((k, n), lambda i: (0, 0)),
            pl.BlockSpec((1, n), lambda i: (0, 0)),
        ],
        out_specs=pl.BlockSpec((QBLK, n), lambda i: (i, 0)),
        out_shape=jax.ShapeDtypeStruct((m, n), jnp.float32),
    )(x_bf, w_bf, b.reshape(1, n))


def _sample_body(x_ref, w_ref, b_ref, rp_ref, idx_ref, wgt_ref):
    i = pl.program_id(0)
    acc = jnp.dot(x_ref[...], w_ref[...],
                  preferred_element_type=jnp.float32) + b_ref[...]
    locx = jnp.clip(rp_ref[:, 0:128] + acc[:, 0:128], 0.0, 1.0)
    locy = jnp.clip(rp_ref[:, 128:256] + acc[:, 128:256], 0.0, 1.0)
    att = acc[:, 256:384]

    m = jnp.max(att, axis=1, keepdims=True)
    e = jnp.exp(att - m)
    gi = lax.broadcasted_iota(jnp.int32, (128, HEADS), 0) // 16
    gj = lax.broadcasted_iota(jnp.int32, (128, HEADS), 1)
    g = (gi == gj).astype(jnp.float32)
    ti = lax.broadcasted_iota(jnp.int32, (HEADS, 128), 0)
    tj = lax.broadcasted_iota(jnp.int32, (HEADS, 128), 1) // 16
    gt = (ti == tj).astype(jnp.float32)
    denom = jnp.dot(e, g, precision=jax.lax.Precision.HIGHEST,
                    preferred_element_type=jnp.float32)
    attn = e * jnp.dot(1.0 / denom, gt,
                       precision=jax.lax.Precision.HIGHEST,
                       preferred_element_type=jnp.float32)

    col = lax.broadcasted_iota(jnp.int32, (QBLK, 128), 1)
    lvl = (col % 16) // 4
    hh = col // 16
    wi = jnp.where(lvl == 0, _SIZES[0],
                   jnp.where(lvl == 1, _SIZES[1],
                             jnp.where(lvl == 2, _SIZES[2], _SIZES[3])))
    start = jnp.where(lvl == 0, _STARTS[0],
                      jnp.where(lvl == 1, _STARTS[1],
                                jnp.where(lvl == 2, _STARTS[2], _STARTS[3])))
    wf = wi.astype(jnp.float32)
    grow = i * QBLK + lax.broadcasted_iota(jnp.int32, (QBLK, 128), 0)
    b = grow // NQ

    gx = (locx * 2.0 - 1.0 + 1.0) * (wf * 0.5) - 0.5
    gy = (locy * 2.0 - 1.0 + 1.0) * (wf * 0.5) - 0.5
    x0f = jnp.floor(gx)
    y0f = jnp.floor(gy)
    wx1 = gx - x0f
    wx0 = 1.0 - wx1
    wy1 = gy - y0f
    wy0 = 1.0 - wy1
    x0 = x0f.astype(jnp.int32)
    y0 = y0f.astype(jnp.int32)
    x1 = x0 + 1
    y1 = y0 + 1

    base = (b * NV + start) * HEADS + hh

    def corner(xi, yi, wgt, c):
        vx = (xi >= 0) & (xi <= wi - 1)
        vy = (yi >= 0) & (yi <= wi - 1)
        xc = jnp.clip(xi, 0, wi - 1)
        yc = jnp.clip(yi, 0, wi - 1)
        rowidx = base + (yc * wi + xc) * HEADS
        wc = jnp.where(vx & vy, wgt * attn, 0.0)
        idx_ref[:, c * 128:(c + 1) * 128] = rowidx
        wgt_ref[:, c * 128:(c + 1) * 128] = wc

    corner(x0, y0, wx0 * wy0, 0)
    corner(x1, y0, wx1 * wy0, 1)
    corner(x0, y1, wx0 * wy1, 2)
    corner(x1, y1, wx1 * wy1, 3)


def _tc_sample_params(q_bf, w_oa_bf, b_oa, rp_wide):
    return pl.pallas_call(
        _sample_body,
        grid=(NBLK,),
        in_specs=[
            pl.BlockSpec((QBLK, EMBED), lambda i: (i, 0)),
            pl.BlockSpec((EMBED, 384), lambda i: (0, 0)),
            pl.BlockSpec((1, 384), lambda i: (0, 0)),
            pl.BlockSpec((QBLK, 256), lambda i: (i, 0)),
        ],
        out_specs=[
            pl.BlockSpec((QBLK, ENTRIES), lambda i: (i, 0)),
            pl.BlockSpec((QBLK, ENTRIES), lambda i: (i, 0)),
        ],
        out_shape=[
            jax.ShapeDtypeStruct((NQB, ENTRIES), jnp.int32),
            jax.ShapeDtypeStruct((NQB, ENTRIES), jnp.float32),
        ],
    )(q_bf, w_oa_bf, b_oa.reshape(1, 384), rp_wide)


_QCHUNK = 2
_ROWS = _QCHUNK * ENTRIES
_NWORK = 32
_PER_W = NQB // _NWORK


def _sc_gather(vtab, idx, wgt):
    mesh = plsc.VectorSubcoreMesh(core_axis_name="c", subcore_axis_name="s")

    @functools.partial(
        pl.kernel,
        mesh=mesh,
        out_type=jax.ShapeDtypeStruct((NQB, EMBED), jnp.float32),
        scratch_types=[
            pltpu.VMEM((_ROWS // 128, 128), jnp.int32),
            pltpu.VMEM((_ROWS,), jnp.float32),
            pltpu.VMEM((_ROWS, HEAD_DIM), jnp.float32),
            pltpu.VMEM((_QCHUNK, EMBED), jnp.float32),
            pltpu.SemaphoreType.DMA,
        ],
        compiler_params=pltpu.CompilerParams(use_tc_tiling_on_sc=False),
    )
    def k(vtab_hbm, idx_hbm, w_hbm, out_hbm, idx_v, w_v, rows_v, out_v, sem):
        wid = lax.axis_index("s") * 2 + lax.axis_index("c")
        base_q = wid * _PER_W

        def step(it, carry):
            q0 = base_q + it * _QCHUNK
            e0 = q0 * ENTRIES
            pltpu.sync_copy(idx_hbm.at[pl.ds(e0 // 128, _ROWS // 128)], idx_v)
            pltpu.sync_copy(w_hbm.at[pl.ds(e0, _ROWS)], w_v)
            copies = [
                pltpu.async_copy(vtab_hbm.at[idx_v.at[j]],
                                 rows_v.at[pl.ds(j * 128, 128)], sem)
                for j in range(_ROWS // 128)
            ]
            for cp in copies:
                cp.wait()
            for q in range(_QCHUNK):
                for h in range(HEADS):
                    def inner(c, accs):
                        a0, a1 = accs
                        kb = q * ENTRIES + c * 128 + h * 16
                        wvec = w_v[pl.ds(kb, 16)]
                        for jj in range(16):
                            wv = wvec[jj]
                            a0 = a0 + wv * rows_v[kb + jj, 0:16]
                            a1 = a1 + wv * rows_v[kb + jj, 16:32]
                        return (a0, a1)

                    z = jnp.zeros((16,), jnp.float32)
                    a0, a1 = lax.fori_loop(0, 4, inner, (z, z))
                    out_v[q, h * 32:h * 32 + 16] = a0
                    out_v[q, h * 32 + 16:h * 32 + 32] = a1
            pltpu.sync_copy(out_v, out_hbm.at[pl.ds(q0, _QCHUNK)])
            return carry

        lax.fori_loop(0, _PER_W // _QCHUNK, step, 0)

    return k(vtab, idx, wgt)


def kernel(query, reference_points, value, W_value, b_value, W_off, b_off,
           W_att, b_att, W_out, b_out, value_spatial_shapes,
           value_level_start_index):
    bf = jnp.bfloat16
    q_bf = query.reshape(NQB, EMBED).astype(bf)
    v_bf = value.reshape(NQB, EMBED).astype(bf)
    rp = reference_points.reshape(NQB, 2)
    rp_wide = jnp.concatenate(
        [jnp.broadcast_to(rp[:, 0:1], (NQB, 128)),
         jnp.broadcast_to(rp[:, 1:2], (NQB, 128))], axis=1)

    perm = jnp.asarray(_ATT_PERM, jnp.int32)
    w_oa = jnp.concatenate([W_off[:, 0::2], W_off[:, 1::2], W_att[:, perm]],
                           axis=1)
    b_oa = jnp.concatenate([b_off[0::2], b_off[1::2], b_att[perm]], axis=0)

    vtab = _tc_matmul(v_bf, W_value.astype(bf),
                      b_value).reshape(NQB * HEADS, HEAD_DIM)
    idx, wgt = _tc_sample_params(q_bf, w_oa.astype(bf), b_oa, rp_wide)
    sc_out = _sc_gather(vtab, idx.reshape(-1, 128), wgt.reshape(-1))
    sc_out = (sc_out.reshape(BS, NQ, HEADS, HEAD_DIM)
              .transpose(0, 2, 1, 3).reshape(NQB, EMBED))
    out = _tc_matmul(sc_out.astype(bf), W_out.astype(bf), b_out)
    return out.reshape(BS, NQ, EMBED)


# speedup vs baseline: 2306.9823x; 1.1378x over previous
import functools

import jax
import jax.numpy as jnp
from jax import lax
from jax.experimental import pallas as pl
from jax.experimental.pallas import tpu as pltpu
from jax.experimental.pallas import tpu_sc as plsc

HEADS = 8
LEVELS = 4
POINTS = 4
EMBED = 256
HEAD_DIM = 32
BS = 2
NQ = 5440
NV = 5440
NQB = BS * NQ
ENTRIES = 4 * HEADS * LEVELS * POINTS
QBLK = 128
NBLK = NQB // QBLK

_SIZES = (64, 32, 16, 8)
_STARTS = (0, 4096, 5120, 5376)

_ATT_PERM = tuple(h * 16 + p * 4 + l
                  for h in range(HEADS)
                  for l in range(LEVELS)
                  for p in range(POINTS))


def _matmul_body(x_ref, w_ref, b_ref, o_ref):
    o_ref[...] = jnp.dot(x_ref[...], w_ref[...],
                         preferred_element_type=jnp.float32) + b_ref[...]


def _tc_matmul(x_bf, w_bf, b):
    m, k = x_bf.shape
    n = w_bf.shape[1]
    return pl.pallas_call(
        _matmul_body,
        grid=(m // QBLK,),
        in_specs=[
            pl.BlockSpec((QBLK, k), lambda i: (i, 0)),
            pl.BlockSpec((k, n), lambda i: (0, 0)),
            pl.BlockSpec((1, n), lambda i: (0, 0)),
        ],
        out_specs=pl.BlockSpec((QBLK, n), lambda i: (i, 0)),
        out_shape=jax.ShapeDtypeStruct((m, n), jnp.float32),
    )(x_bf, w_bf, b.reshape(1, n))


def _sample_body(x_ref, w_ref, b_ref, rp_ref, idx_ref, wgt_ref):
    i = pl.program_id(0)
    acc = jnp.dot(x_ref[...], w_ref[...],
                  preferred_element_type=jnp.float32) + b_ref[...]
    locx = jnp.clip(rp_ref[:, 0:128] + acc[:, 0:128], 0.0, 1.0)
    locy = jnp.clip(rp_ref[:, 128:256] + acc[:, 128:256], 0.0, 1.0)
    att = acc[:, 256:384]

    m = jnp.max(att, axis=1, keepdims=True)
    e = jnp.exp(att - m)
    gi = lax.broadcasted_iota(jnp.int32, (128, HEADS), 0) // 16
    gj = lax.broadcasted_iota(jnp.int32, (128, HEADS), 1)
    g = (gi == gj).astype(jnp.float32)
    ti = lax.broadcasted_iota(jnp.int32, (HEADS, 128), 0)
    tj = lax.broadcasted_iota(jnp.int32, (HEADS, 128), 1) // 16
    gt = (ti == tj).astype(jnp.float32)
    denom = jnp.dot(e, g, precision=jax.lax.Precision.HIGHEST,
                    preferred_element_type=jnp.float32)
    attn = e * jnp.dot(1.0 / denom, gt,
                       precision=jax.lax.Precision.HIGHEST,
                       preferred_element_type=jnp.float32)

    col = lax.broadcasted_iota(jnp.int32, (QBLK, 128), 1)
    lvl = (col % 16) // 4
    hh = col // 16
    wi = jnp.where(lvl == 0, _SIZES[0],
                   jnp.where(lvl == 1, _SIZES[1],
                             jnp.where(lvl == 2, _SIZES[2], _SIZES[3])))
    start = jnp.where(lvl == 0, _STARTS[0],
                      jnp.where(lvl == 1, _STARTS[1],
                                jnp.where(lvl == 2, _STARTS[2], _STARTS[3])))
    wf = wi.astype(jnp.float32)
    grow = i * QBLK + lax.broadcasted_iota(jnp.int32, (QBLK, 128), 0)
    b = grow // NQ

    gx = (locx * 2.0 - 1.0 + 1.0) * (wf * 0.5) - 0.5
    gy = (locy * 2.0 - 1.0 + 1.0) * (wf * 0.5) - 0.5
    x0f = jnp.floor(gx)
    y0f = jnp.floor(gy)
    wx1 = gx - x0f
    wx0 = 1.0 - wx1
    wy1 = gy - y0f
    wy0 = 1.0 - wy1
    x0 = x0f.astype(jnp.int32)
    y0 = y0f.astype(jnp.int32)
    x1 = x0 + 1
    y1 = y0 + 1

    base = (b * NV + start) * HEADS + hh

    def corner(xi, yi, wgt, c):
        vx = (xi >= 0) & (xi <= wi - 1)
        vy = (yi >= 0) & (yi <= wi - 1)
        xc = jnp.clip(xi, 0, wi - 1)
        yc = jnp.clip(yi, 0, wi - 1)
        rowidx = base + (yc * wi + xc) * HEADS
        wc = jnp.where(vx & vy, wgt * attn, 0.0)
        idx_ref[:, c * 128:(c + 1) * 128] = rowidx
        wgt_ref[:, c * 128:(c + 1) * 128] = wc

    corner(x0, y0, wx0 * wy0, 0)
    corner(x1, y0, wx1 * wy0, 1)
    corner(x0, y1, wx0 * wy1, 2)
    corner(x1, y1, wx1 * wy1, 3)


def _tc_sample_params(q_bf, w_oa_bf, b_oa, rp_wide):
    return pl.pallas_call(
        _sample_body,
        grid=(NBLK,),
        in_specs=[
            pl.BlockSpec((QBLK, EMBED), lambda i: (i, 0)),
            pl.BlockSpec((EMBED, 384), lambda i: (0, 0)),
            pl.BlockSpec((1, 384), lambda i: (0, 0)),
            pl.BlockSpec((QBLK, 256), lambda i: (i, 0)),
        ],
        out_specs=[
            pl.BlockSpec((QBLK, ENTRIES), lambda i: (i, 0)),
            pl.BlockSpec((QBLK, ENTRIES), lambda i: (i, 0)),
        ],
        out_shape=[
            jax.ShapeDtypeStruct((NQB, ENTRIES), jnp.int32),
            jax.ShapeDtypeStruct((NQB, ENTRIES), jnp.float32),
        ],
    )(q_bf, w_oa_bf, b_oa.reshape(1, 384), rp_wide)


_QCHUNK = 2
_ROWS = _QCHUNK * ENTRIES
_NWORK = 32
_PER_W = NQB // _NWORK


_NSTEP = _PER_W // _QCHUNK


def _sc_gather(vtab, idx, wgt):
    mesh = plsc.VectorSubcoreMesh(core_axis_name="c", subcore_axis_name="s")

    @functools.partial(
        pl.kernel,
        mesh=mesh,
        out_type=jax.ShapeDtypeStruct((NQB, EMBED), jnp.float32),
        scratch_types=[
            pltpu.VMEM((2, _ROWS // 128, 128), jnp.int32),
            pltpu.VMEM((2, _ROWS), jnp.float32),
            pltpu.VMEM((2, _ROWS, HEAD_DIM), jnp.float32),
            pltpu.VMEM((_QCHUNK, EMBED), jnp.float32),
            (pltpu.SemaphoreType.DMA, pltpu.SemaphoreType.DMA),
        ],
        compiler_params=pltpu.CompilerParams(use_tc_tiling_on_sc=False),
    )
    def k(vtab_hbm, idx_hbm, w_hbm, out_hbm, idx_v, w_v, rows_v, out_v, sems):
        wid = lax.axis_index("s") * 2 + lax.axis_index("c")
        base_q = wid * _PER_W

        def fetch(step_i, buf):
            q0 = base_q + step_i * _QCHUNK
            e0 = q0 * ENTRIES
            pltpu.sync_copy(idx_hbm.at[pl.ds(e0 // 128, _ROWS // 128)],
                            idx_v.at[buf])
            pltpu.sync_copy(w_hbm.at[pl.ds(e0, _ROWS)], w_v.at[buf])
            for j in range(_ROWS // 128):
                pltpu.async_copy(vtab_hbm.at[idx_v.at[buf].at[j]],
                                 rows_v.at[buf].at[pl.ds(j * 128, 128)],
                                 sems[buf])

        def drain_compute(step_i, buf):
            for j in range(_ROWS // 128):
                pltpu.make_async_copy(
                    vtab_hbm.at[idx_v.at[buf].at[j]],
                    rows_v.at[buf].at[pl.ds(j * 128, 128)],
                    sems[buf]).wait()
            rbuf = rows_v.at[buf]
            for q in range(_QCHUNK):
                for h in range(HEADS):
                    def inner(c, accs):
                        a0, a1 = accs
                        kb = q * ENTRIES + c * 128 + h * 16
                        wvec = w_v[buf, pl.ds(kb, 16)]
                        for jj in range(16):
                            wv = wvec[jj]
                            a0 = a0 + wv * rbuf[kb + jj, 0:16]
                            a1 = a1 + wv * rbuf[kb + jj, 16:32]
                        return (a0, a1)

                    z = jnp.zeros((16,), jnp.float32)
                    a0, a1 = lax.fori_loop(0, 4, inner, (z, z))
                    out_v[q, h * 32:h * 32 + 16] = a0
                    out_v[q, h * 32 + 16:h * 32 + 32] = a1
            q0 = base_q + step_i * _QCHUNK
            pltpu.sync_copy(out_v, out_hbm.at[pl.ds(q0, _QCHUNK)])

        fetch(0, 0)

        def dbl(d, carry):
            g = d * 2
            fetch(g + 1, 1)
            drain_compute(g, 0)

            @pl.when(g + 2 < _NSTEP)
            def _():
                fetch(g + 2, 0)

            drain_compute(g + 1, 1)
            return carry

        lax.fori_loop(0, _NSTEP // 2, dbl, 0)

    return k(vtab, idx, wgt)


def kernel(query, reference_points, value, W_value, b_value, W_off, b_off,
           W_att, b_att, W_out, b_out, value_spatial_shapes,
           value_level_start_index):
    bf = jnp.bfloat16
    q_bf = query.reshape(NQB, EMBED).astype(bf)
    v_bf = value.reshape(NQB, EMBED).astype(bf)
    rp = reference_points.reshape(NQB, 2)
    rp_wide = jnp.concatenate(
        [jnp.broadcast_to(rp[:, 0:1], (NQB, 128)),
         jnp.broadcast_to(rp[:, 1:2], (NQB, 128))], axis=1)

    perm = jnp.asarray(_ATT_PERM, jnp.int32)
    w_oa = jnp.concatenate([W_off[:, 0::2], W_off[:, 1::2], W_att[:, perm]],
                           axis=1)
    b_oa = jnp.concatenate([b_off[0::2], b_off[1::2], b_att[perm]], axis=0)

    vtab = _tc_matmul(v_bf, W_value.astype(bf),
                      b_value).reshape(NQB * HEADS, HEAD_DIM)
    idx, wgt = _tc_sample_params(q_bf, w_oa.astype(bf), b_oa, rp_wide)
    sc_out = _sc_gather(vtab, idx.reshape(-1, 128), wgt.reshape(-1))
    sc_out = (sc_out.reshape(BS, NQ, HEADS, HEAD_DIM)
              .transpose(0, 2, 1, 3).reshape(NQB, EMBED))
    out = _tc_matmul(sc_out.astype(bf), W_out.astype(bf), b_out)
    return out.reshape(BS, NQ, EMBED)


# speedup vs baseline: 2720.6933x; 1.1793x over previous
import functools

import jax
import jax.numpy as jnp
from jax import lax
from jax.experimental import pallas as pl
from jax.experimental.pallas import tpu as pltpu
from jax.experimental.pallas import tpu_sc as plsc

HEADS = 8
LEVELS = 4
POINTS = 4
EMBED = 256
HEAD_DIM = 32
BS = 2
NQ = 5440
NV = 5440
NQB = BS * NQ
ENTRIES = 4 * HEADS * LEVELS * POINTS
QBLK = 128
NBLK = NQB // QBLK

_SIZES = (64, 32, 16, 8)
_STARTS = (0, 4096, 5120, 5376)

_ATT_PERM = tuple(h * 16 + p * 4 + l
                  for h in range(HEADS)
                  for l in range(LEVELS)
                  for p in range(POINTS))


def _matmul_body(x_ref, w_ref, b_ref, o_ref):
    o_ref[...] = jnp.dot(x_ref[...], w_ref[...],
                         preferred_element_type=jnp.float32) + b_ref[...]


def _tc_matmul(x_bf, w_bf, b):
    m, k = x_bf.shape
    n = w_bf.shape[1]
    return pl.pallas_call(
        _matmul_body,
        grid=(m // QBLK,),
        in_specs=[
            pl.BlockSpec((QBLK, k), lambda i: (i, 0)),
            pl.BlockSpec((k, n), lambda i: (0, 0)),
            pl.BlockSpec((1, n), lambda i: (0, 0)),
        ],
        out_specs=pl.BlockSpec((QBLK, n), lambda i: (i, 0)),
        out_shape=jax.ShapeDtypeStruct((m, n), jnp.float32),
    )(x_bf, w_bf, b.reshape(1, n))


def _sample_body(x_ref, w_ref, b_ref, rp_ref, idx_ref, wgt_ref):
    i = pl.program_id(0)
    acc = jnp.dot(x_ref[...], w_ref[...],
                  preferred_element_type=jnp.float32) + b_ref[...]
    locx = jnp.clip(rp_ref[:, 0:128] + acc[:, 0:128], 0.0, 1.0)
    locy = jnp.clip(rp_ref[:, 128:256] + acc[:, 128:256], 0.0, 1.0)
    att = acc[:, 256:384]

    m = jnp.max(att, axis=1, keepdims=True)
    e = jnp.exp(att - m)
    gi = lax.broadcasted_iota(jnp.int32, (128, HEADS), 0) // 16
    gj = lax.broadcasted_iota(jnp.int32, (128, HEADS), 1)
    g = (gi == gj).astype(jnp.float32)
    ti = lax.broadcasted_iota(jnp.int32, (HEADS, 128), 0)
    tj = lax.broadcasted_iota(jnp.int32, (HEADS, 128), 1) // 16
    gt = (ti == tj).astype(jnp.float32)
    denom = jnp.dot(e, g, precision=jax.lax.Precision.HIGHEST,
                    preferred_element_type=jnp.float32)
    attn = e * jnp.dot(1.0 / denom, gt,
                       precision=jax.lax.Precision.HIGHEST,
                       preferred_element_type=jnp.float32)

    col = lax.broadcasted_iota(jnp.int32, (QBLK, 128), 1)
    lvl = (col % 16) // 4
    hh = col // 16
    wi = jnp.where(lvl == 0, _SIZES[0],
                   jnp.where(lvl == 1, _SIZES[1],
                             jnp.where(lvl == 2, _SIZES[2], _SIZES[3])))
    start = jnp.where(lvl == 0, _STARTS[0],
                      jnp.where(lvl == 1, _STARTS[1],
                                jnp.where(lvl == 2, _STARTS[2], _STARTS[3])))
    wf = wi.astype(jnp.float32)
    grow = i * QBLK + lax.broadcasted_iota(jnp.int32, (QBLK, 128), 0)
    b = grow // NQ

    gx = (locx * 2.0 - 1.0 + 1.0) * (wf * 0.5) - 0.5
    gy = (locy * 2.0 - 1.0 + 1.0) * (wf * 0.5) - 0.5
    x0f = jnp.floor(gx)
    y0f = jnp.floor(gy)
    wx1 = gx - x0f
    wx0 = 1.0 - wx1
    wy1 = gy - y0f
    wy0 = 1.0 - wy1
    x0 = x0f.astype(jnp.int32)
    y0 = y0f.astype(jnp.int32)
    x1 = x0 + 1
    y1 = y0 + 1

    base = (b * NV + start) * HEADS + hh

    def corner(xi, yi, wgt, c):
        vx = (xi >= 0) & (xi <= wi - 1)
        vy = (yi >= 0) & (yi <= wi - 1)
        xc = jnp.clip(xi, 0, wi - 1)
        yc = jnp.clip(yi, 0, wi - 1)
        rowidx = base + (yc * wi + xc) * HEADS
        wc = jnp.where(vx & vy, wgt * attn, 0.0)
        idx_ref[:, c * 128:(c + 1) * 128] = rowidx
        wgt_ref[:, c * 128:(c + 1) * 128] = wc

    corner(x0, y0, wx0 * wy0, 0)
    corner(x1, y0, wx1 * wy0, 1)
    corner(x0, y1, wx0 * wy1, 2)
    corner(x1, y1, wx1 * wy1, 3)


def _tc_sample_params(q_bf, w_oa_bf, b_oa, rp_wide):
    return pl.pallas_call(
        _sample_body,
        grid=(NBLK,),
        in_specs=[
            pl.BlockSpec((QBLK, EMBED), lambda i: (i, 0)),
            pl.BlockSpec((EMBED, 384), lambda i: (0, 0)),
            pl.BlockSpec((1, 384), lambda i: (0, 0)),
            pl.BlockSpec((QBLK, 256), lambda i: (i, 0)),
        ],
        out_specs=[
            pl.BlockSpec((QBLK, ENTRIES), lambda i: (i, 0)),
            pl.BlockSpec((QBLK, ENTRIES), lambda i: (i, 0)),
        ],
        out_shape=[
            jax.ShapeDtypeStruct((NQB, ENTRIES), jnp.int32),
            jax.ShapeDtypeStruct((NQB, ENTRIES), jnp.float32),
        ],
    )(q_bf, w_oa_bf, b_oa.reshape(1, 384), rp_wide)


_QCHUNK = 2
_ROWS = _QCHUNK * ENTRIES
_NWORK = 32
_PER_W = NQB // _NWORK


_NSTEP = _PER_W // _QCHUNK


def _sc_gather(vtab, idx, wgt):
    mesh = plsc.VectorSubcoreMesh(core_axis_name="c", subcore_axis_name="s")

    @functools.partial(
        pl.kernel,
        mesh=mesh,
        out_type=jax.ShapeDtypeStruct((NQB, EMBED), jnp.float32),
        scratch_types=[
            pltpu.VMEM((2, _ROWS // 128, 128), jnp.int32),
            pltpu.VMEM((2, _ROWS), jnp.float32),
            pltpu.VMEM((2, _ROWS, HEAD_DIM), jnp.float32),
            pltpu.VMEM((_QCHUNK, EMBED), jnp.float32),
            (pltpu.SemaphoreType.DMA, pltpu.SemaphoreType.DMA),
        ],
        compiler_params=pltpu.CompilerParams(use_tc_tiling_on_sc=False),
    )
    def k(vtab_hbm, idx_hbm, w_hbm, out_hbm, idx_v, w_v, rows_v, out_v, sems):
        wid = lax.axis_index("s") * 2 + lax.axis_index("c")
        base_q = wid * _PER_W

        def fetch(step_i, buf):
            q0 = base_q + step_i * _QCHUNK
            e0 = q0 * ENTRIES
            pltpu.sync_copy(idx_hbm.at[pl.ds(e0 // 128, _ROWS // 128)],
                            idx_v.at[buf])
            pltpu.sync_copy(w_hbm.at[pl.ds(e0, _ROWS)], w_v.at[buf])
            for j in range(_ROWS // 128):
                pltpu.async_copy(vtab_hbm.at[idx_v.at[buf].at[j]],
                                 rows_v.at[buf].at[pl.ds(j * 128, 128)],
                                 sems[buf])

        def drain_compute(step_i, buf):
            for j in range(_ROWS // 128):
                pltpu.make_async_copy(
                    vtab_hbm.at[idx_v.at[buf].at[j]],
                    rows_v.at[buf].at[pl.ds(j * 128, 128)],
                    sems[buf]).wait()
            rbuf = rows_v.at[buf]
            for q in range(_QCHUNK):
                for h in range(HEADS):
                    def inner(c, accs):
                        aa = list(accs[0:4])
                        bb = list(accs[4:8])
                        kb = q * ENTRIES + c * 128 + h * 16
                        wvec = w_v[buf, pl.ds(kb, 16)]
                        for jj in range(16):
                            wv = wvec[jj]
                            aa[jj % 4] = aa[jj % 4] + wv * rbuf[kb + jj, 0:16]
                            bb[jj % 4] = bb[jj % 4] + wv * rbuf[kb + jj, 16:32]
                        return tuple(aa) + tuple(bb)

                    z = jnp.zeros((16,), jnp.float32)
                    accs = lax.fori_loop(0, 4, inner, (z,) * 8)
                    a0 = (accs[0] + accs[1]) + (accs[2] + accs[3])
                    a1 = (accs[4] + accs[5]) + (accs[6] + accs[7])
                    out_v[q, h * 32:h * 32 + 16] = a0
                    out_v[q, h * 32 + 16:h * 32 + 32] = a1
            q0 = base_q + step_i * _QCHUNK
            pltpu.sync_copy(out_v, out_hbm.at[pl.ds(q0, _QCHUNK)])

        fetch(0, 0)

        def dbl(d, carry):
            g = d * 2
            fetch(g + 1, 1)
            drain_compute(g, 0)

            @pl.when(g + 2 < _NSTEP)
            def _():
                fetch(g + 2, 0)

            drain_compute(g + 1, 1)
            return carry

        lax.fori_loop(0, _NSTEP // 2, dbl, 0)

    return k(vtab, idx, wgt)


def kernel(query, reference_points, value, W_value, b_value, W_off, b_off,
           W_att, b_att, W_out, b_out, value_spatial_shapes,
           value_level_start_index):
    bf = jnp.bfloat16
    q_bf = query.reshape(NQB, EMBED).astype(bf)
    v_bf = value.reshape(NQB, EMBED).astype(bf)
    rp = reference_points.reshape(NQB, 2)
    rp_wide = jnp.concatenate(
        [jnp.broadcast_to(rp[:, 0:1], (NQB, 128)),
         jnp.broadcast_to(rp[:, 1:2], (NQB, 128))], axis=1)

    perm = jnp.asarray(_ATT_PERM, jnp.int32)
    w_oa = jnp.concatenate([W_off[:, 0::2], W_off[:, 1::2], W_att[:, perm]],
                           axis=1)
    b_oa = jnp.concatenate([b_off[0::2], b_off[1::2], b_att[perm]], axis=0)

    vtab = _tc_matmul(v_bf, W_value.astype(bf),
                      b_value).reshape(NQB * HEADS, HEAD_DIM)
    idx, wgt = _tc_sample_params(q_bf, w_oa.astype(bf), b_oa, rp_wide)
    sc_out = _sc_gather(vtab, idx.reshape(-1, 128), wgt.reshape(-1))
    sc_out = (sc_out.reshape(BS, NQ, HEADS, HEAD_DIM)
              .transpose(0, 2, 1, 3).reshape(NQB, EMBED))
    out = _tc_matmul(sc_out.astype(bf), W_out.astype(bf), b_out)
    return out.reshape(BS, NQ, EMBED)


# speedup vs baseline: 2915.3357x; 1.0715x over previous
import functools

import jax
import jax.numpy as jnp
from jax import lax
from jax.experimental import pallas as pl
from jax.experimental.pallas import tpu as pltpu
from jax.experimental.pallas import tpu_sc as plsc

HEADS = 8
LEVELS = 4
POINTS = 4
EMBED = 256
HEAD_DIM = 32
BS = 2
NQ = 5440
NV = 5440
NQB = BS * NQ
ENTRIES = 4 * HEADS * LEVELS * POINTS
QBLK = 128
NBLK = NQB // QBLK

_SIZES = (64, 32, 16, 8)
_STARTS = (0, 4096, 5120, 5376)

_ATT_PERM = tuple(h * 16 + p * 4 + l
                  for h in range(HEADS)
                  for l in range(LEVELS)
                  for p in range(POINTS))


def _matmul_body(x_ref, w_ref, b_ref, o_ref):
    o_ref[...] = jnp.dot(x_ref[...], w_ref[...],
                         preferred_element_type=jnp.float32) + b_ref[...]


def _tc_matmul(x_bf, w_bf, b):
    m, k = x_bf.shape
    n = w_bf.shape[1]
    return pl.pallas_call(
        _matmul_body,
        grid=(m // QBLK,),
        in_specs=[
            pl.BlockSpec((QBLK, k), lambda i: (i, 0)),
            pl.BlockSpec((k, n), lambda i: (0, 0)),
            pl.BlockSpec((1, n), lambda i: (0, 0)),
        ],
        out_specs=pl.BlockSpec((QBLK, n), lambda i: (i, 0)),
        out_shape=jax.ShapeDtypeStruct((m, n), jnp.float32),
    )(x_bf, w_bf, b.reshape(1, n))


def _sample_body(x_ref, w_ref, b_ref, rp_ref, idx_ref, wgt_ref):
    i = pl.program_id(0)
    acc = jnp.dot(x_ref[...], w_ref[...],
                  preferred_element_type=jnp.float32) + b_ref[...]
    locx = jnp.clip(rp_ref[:, 0:128] + acc[:, 0:128], 0.0, 1.0)
    locy = jnp.clip(rp_ref[:, 128:256] + acc[:, 128:256], 0.0, 1.0)
    att = acc[:, 256:384]

    m = jnp.max(att, axis=1, keepdims=True)
    e = jnp.exp(att - m)
    gi = lax.broadcasted_iota(jnp.int32, (128, HEADS), 0) // 16
    gj = lax.broadcasted_iota(jnp.int32, (128, HEADS), 1)
    g = (gi == gj).astype(jnp.float32)
    ti = lax.broadcasted_iota(jnp.int32, (HEADS, 128), 0)
    tj = lax.broadcasted_iota(jnp.int32, (HEADS, 128), 1) // 16
    gt = (ti == tj).astype(jnp.float32)
    denom = jnp.dot(e, g, precision=jax.lax.Precision.HIGHEST,
                    preferred_element_type=jnp.float32)
    attn = e * jnp.dot(1.0 / denom, gt,
                       precision=jax.lax.Precision.HIGHEST,
                       preferred_element_type=jnp.float32)

    col = lax.broadcasted_iota(jnp.int32, (QBLK, 128), 1)
    lvl = (col % 16) // 4
    hh = col // 16
    wi = jnp.where(lvl == 0, _SIZES[0],
                   jnp.where(lvl == 1, _SIZES[1],
                             jnp.where(lvl == 2, _SIZES[2], _SIZES[3])))
    start = jnp.where(lvl == 0, _STARTS[0],
                      jnp.where(lvl == 1, _STARTS[1],
                                jnp.where(lvl == 2, _STARTS[2], _STARTS[3])))
    wf = wi.astype(jnp.float32)
    grow = i * QBLK + lax.broadcasted_iota(jnp.int32, (QBLK, 128), 0)
    b = grow // NQ

    gx = (locx * 2.0 - 1.0 + 1.0) * (wf * 0.5) - 0.5
    gy = (locy * 2.0 - 1.0 + 1.0) * (wf * 0.5) - 0.5
    x0f = jnp.floor(gx)
    y0f = jnp.floor(gy)
    wx1 = gx - x0f
    wx0 = 1.0 - wx1
    wy1 = gy - y0f
    wy0 = 1.0 - wy1
    x0 = x0f.astype(jnp.int32)
    y0 = y0f.astype(jnp.int32)
    x1 = x0 + 1
    y1 = y0 + 1

    base = (b * NV + start) * HEADS + hh

    def corner(xi, yi, wgt, c):
        vx = (xi >= 0) & (xi <= wi - 1)
        vy = (yi >= 0) & (yi <= wi - 1)
        xc = jnp.clip(xi, 0, wi - 1)
        yc = jnp.clip(yi, 0, wi - 1)
        rowidx = base + (yc * wi + xc) * HEADS
        wc = jnp.where(vx & vy, wgt * attn, 0.0)
        idx_ref[:, c * 128:(c + 1) * 128] = rowidx
        wgt_ref[:, c * 128:(c + 1) * 128] = wc

    corner(x0, y0, wx0 * wy0, 0)
    corner(x1, y0, wx1 * wy0, 1)
    corner(x0, y1, wx0 * wy1, 2)
    corner(x1, y1, wx1 * wy1, 3)


def _tc_sample_params(q_bf, w_oa_bf, b_oa, rp_wide):
    return pl.pallas_call(
        _sample_body,
        grid=(NBLK,),
        in_specs=[
            pl.BlockSpec((QBLK, EMBED), lambda i: (i, 0)),
            pl.BlockSpec((EMBED, 384), lambda i: (0, 0)),
            pl.BlockSpec((1, 384), lambda i: (0, 0)),
            pl.BlockSpec((QBLK, 256), lambda i: (i, 0)),
        ],
        out_specs=[
            pl.BlockSpec((QBLK, ENTRIES), lambda i: (i, 0)),
            pl.BlockSpec((QBLK, ENTRIES), lambda i: (i, 0)),
        ],
        out_shape=[
            jax.ShapeDtypeStruct((NQB, ENTRIES), jnp.int32),
            jax.ShapeDtypeStruct((NQB, ENTRIES), jnp.float32),
        ],
    )(q_bf, w_oa_bf, b_oa.reshape(1, 384), rp_wide)


_QCHUNK = 2
_ROWS = _QCHUNK * ENTRIES
_NWORK = 32
_PER_W = NQB // _NWORK


_NSTEP = _PER_W // _QCHUNK


def _sc_gather(vtab, idx, wgt):
    mesh = plsc.VectorSubcoreMesh(core_axis_name="c", subcore_axis_name="s")

    @functools.partial(
        pl.kernel,
        mesh=mesh,
        out_type=jax.ShapeDtypeStruct((NQB, EMBED), jnp.float32),
        scratch_types=[
            pltpu.VMEM((2, _ROWS // 128, 128), jnp.int32),
            pltpu.VMEM((2, _ROWS), jnp.float32),
            pltpu.VMEM((2, _ROWS, HEAD_DIM), jnp.float32),
            pltpu.VMEM((_QCHUNK, EMBED), jnp.float32),
            (pltpu.SemaphoreType.DMA, pltpu.SemaphoreType.DMA),
            (pltpu.SemaphoreType.DMA, pltpu.SemaphoreType.DMA),
        ],
        compiler_params=pltpu.CompilerParams(use_tc_tiling_on_sc=False),
    )
    def k(vtab_hbm, idx_hbm, w_hbm, out_hbm, idx_v, w_v, rows_v, out_v,
          sems, isems):
        wid = lax.axis_index("s") * 2 + lax.axis_index("c")
        base_q = wid * _PER_W

        def stage(step_i, buf):
            q0 = base_q + step_i * _QCHUNK
            e0 = q0 * ENTRIES
            pltpu.async_copy(idx_hbm.at[pl.ds(e0 // 128, _ROWS // 128)],
                             idx_v.at[buf], isems[buf])
            pltpu.async_copy(w_hbm.at[pl.ds(e0, _ROWS)], w_v.at[buf],
                             isems[buf])

        def stage_wait(step_i, buf):
            q0 = base_q + step_i * _QCHUNK
            e0 = q0 * ENTRIES
            pltpu.make_async_copy(
                idx_hbm.at[pl.ds(e0 // 128, _ROWS // 128)],
                idx_v.at[buf], isems[buf]).wait()
            pltpu.make_async_copy(
                w_hbm.at[pl.ds(e0, _ROWS)], w_v.at[buf], isems[buf]).wait()

        def fire(step_i, buf):
            for j in range(_ROWS // 128):
                pltpu.async_copy(vtab_hbm.at[idx_v.at[buf].at[j]],
                                 rows_v.at[buf].at[pl.ds(j * 128, 128)],
                                 sems[buf])

        def drain_compute(step_i, buf):
            for j in range(_ROWS // 128):
                pltpu.make_async_copy(
                    vtab_hbm.at[idx_v.at[buf].at[j]],
                    rows_v.at[buf].at[pl.ds(j * 128, 128)],
                    sems[buf]).wait()
            rbuf = rows_v.at[buf]
            for q in range(_QCHUNK):
                for h in range(HEADS):
                    def inner(c, accs):
                        aa = list(accs[0:4])
                        bb = list(accs[4:8])
                        kb = q * ENTRIES + c * 128 + h * 16
                        wvec = w_v[buf, pl.ds(kb, 16)]
                        for jj in range(16):
                            wv = wvec[jj]
                            aa[jj % 4] = aa[jj % 4] + wv * rbuf[kb + jj, 0:16]
                            bb[jj % 4] = bb[jj % 4] + wv * rbuf[kb + jj, 16:32]
                        return tuple(aa) + tuple(bb)

                    z = jnp.zeros((16,), jnp.float32)
                    accs = lax.fori_loop(0, 4, inner, (z,) * 8)
                    a0 = (accs[0] + accs[1]) + (accs[2] + accs[3])
                    a1 = (accs[4] + accs[5]) + (accs[6] + accs[7])
                    out_v[q, h * 32:h * 32 + 16] = a0
                    out_v[q, h * 32 + 16:h * 32 + 32] = a1
            q0 = base_q + step_i * _QCHUNK
            pltpu.sync_copy(out_v, out_hbm.at[pl.ds(q0, _QCHUNK)])

        stage(0, 0)
        stage_wait(0, 0)
        fire(0, 0)
        stage(1, 1)

        def phase(g, buf):
            @pl.when(g + 1 < _NSTEP)
            def _():
                stage_wait(g + 1, buf ^ 1)
                fire(g + 1, buf ^ 1)

            drain_compute(g, buf)

            @pl.when(g + 2 < _NSTEP)
            def _():
                stage(g + 2, buf)

        def dbl(d, carry):
            g = d * 2
            phase(g, 0)
            phase(g + 1, 1)
            return carry

        lax.fori_loop(0, _NSTEP // 2, dbl, 0)

    return k(vtab, idx, wgt)


def kernel(query, reference_points, value, W_value, b_value, W_off, b_off,
           W_att, b_att, W_out, b_out, value_spatial_shapes,
           value_level_start_index):
    bf = jnp.bfloat16
    q_bf = query.reshape(NQB, EMBED).astype(bf)
    v_bf = value.reshape(NQB, EMBED).astype(bf)
    rp = reference_points.reshape(NQB, 2)
    rp_wide = jnp.concatenate(
        [jnp.broadcast_to(rp[:, 0:1], (NQB, 128)),
         jnp.broadcast_to(rp[:, 1:2], (NQB, 128))], axis=1)

    perm = jnp.asarray(_ATT_PERM, jnp.int32)
    w_oa = jnp.concatenate([W_off[:, 0::2], W_off[:, 1::2], W_att[:, perm]],
                           axis=1)
    b_oa = jnp.concatenate([b_off[0::2], b_off[1::2], b_att[perm]], axis=0)

    vtab = _tc_matmul(v_bf, W_value.astype(bf),
                      b_value).reshape(NQB * HEADS, HEAD_DIM)
    idx, wgt = _tc_sample_params(q_bf, w_oa.astype(bf), b_oa, rp_wide)
    sc_out = _sc_gather(vtab, idx.reshape(-1, 128), wgt.reshape(-1))
    sc_out = (sc_out.reshape(BS, NQ, HEADS, HEAD_DIM)
              .transpose(0, 2, 1, 3).reshape(NQB, EMBED))
    out = _tc_matmul(sc_out.astype(bf), W_out.astype(bf), b_out)
    return out.reshape(BS, NQ, EMBED)


# speedup vs baseline: 3241.9000x; 1.1120x over previous
import functools

import jax
import jax.numpy as jnp
from jax import lax
from jax.experimental import pallas as pl
from jax.experimental.pallas import tpu as pltpu
from jax.experimental.pallas import tpu_sc as plsc

HEADS = 8
LEVELS = 4
POINTS = 4
EMBED = 256
HEAD_DIM = 32
BS = 2
NQ = 5440
NV = 5440
NQB = BS * NQ
ENTRIES = 4 * HEADS * LEVELS * POINTS
QBLK = 128
NBLK = NQB // QBLK

_SIZES = (64, 32, 16, 8)
_STARTS = (0, 4096, 5120, 5376)

_ATT_PERM = tuple(h * 16 + p * 4 + l
                  for h in range(HEADS)
                  for l in range(LEVELS)
                  for p in range(POINTS))


def _matmul_body(x_ref, w_ref, b_ref, o_ref):
    o_ref[...] = jnp.dot(x_ref[...].astype(jnp.bfloat16),
                         w_ref[...].astype(jnp.bfloat16),
                         preferred_element_type=jnp.float32) + b_ref[...]


def _tc_matmul(x_bf, w_bf, b):
    m, k = x_bf.shape
    n = w_bf.shape[1]
    return pl.pallas_call(
        _matmul_body,
        grid=(m // QBLK,),
        in_specs=[
            pl.BlockSpec((QBLK, k), lambda i: (i, 0)),
            pl.BlockSpec((k, n), lambda i: (0, 0)),
            pl.BlockSpec((1, n), lambda i: (0, 0)),
        ],
        out_specs=pl.BlockSpec((QBLK, n), lambda i: (i, 0)),
        out_shape=jax.ShapeDtypeStruct((m, n), jnp.float32),
    )(x_bf, w_bf, b.reshape(1, n))


def _sample_body(x_ref, w_ref, b_ref, rp_ref, idx_ref, wgt_ref):
    i = pl.program_id(0)
    acc = jnp.dot(x_ref[...].astype(jnp.bfloat16),
                  w_ref[...].astype(jnp.bfloat16),
                  preferred_element_type=jnp.float32) + b_ref[...]
    locx = jnp.clip(rp_ref[:, 0:128] + acc[:, 0:128], 0.0, 1.0)
    locy = jnp.clip(rp_ref[:, 128:256] + acc[:, 128:256], 0.0, 1.0)
    att = acc[:, 256:384]

    m = jnp.max(att, axis=1, keepdims=True)
    e = jnp.exp(att - m)
    gi = lax.broadcasted_iota(jnp.int32, (128, HEADS), 0) // 16
    gj = lax.broadcasted_iota(jnp.int32, (128, HEADS), 1)
    g = (gi == gj).astype(jnp.float32)
    ti = lax.broadcasted_iota(jnp.int32, (HEADS, 128), 0)
    tj = lax.broadcasted_iota(jnp.int32, (HEADS, 128), 1) // 16
    gt = (ti == tj).astype(jnp.float32)
    denom = jnp.dot(e, g, precision=jax.lax.Precision.HIGHEST,
                    preferred_element_type=jnp.float32)
    attn = e * jnp.dot(1.0 / denom, gt,
                       precision=jax.lax.Precision.HIGHEST,
                       preferred_element_type=jnp.float32)

    col = lax.broadcasted_iota(jnp.int32, (QBLK, 128), 1)
    lvl = (col % 16) // 4
    hh = col // 16
    wi = jnp.where(lvl == 0, _SIZES[0],
                   jnp.where(lvl == 1, _SIZES[1],
                             jnp.where(lvl == 2, _SIZES[2], _SIZES[3])))
    start = jnp.where(lvl == 0, _STARTS[0],
                      jnp.where(lvl == 1, _STARTS[1],
                                jnp.where(lvl == 2, _STARTS[2], _STARTS[3])))
    wf = wi.astype(jnp.float32)
    grow = i * QBLK + lax.broadcasted_iota(jnp.int32, (QBLK, 128), 0)
    b = grow // NQ

    gx = (locx * 2.0 - 1.0 + 1.0) * (wf * 0.5) - 0.5
    gy = (locy * 2.0 - 1.0 + 1.0) * (wf * 0.5) - 0.5
    x0f = jnp.floor(gx)
    y0f = jnp.floor(gy)
    wx1 = gx - x0f
    wx0 = 1.0 - wx1
    wy1 = gy - y0f
    wy0 = 1.0 - wy1
    x0 = x0f.astype(jnp.int32)
    y0 = y0f.astype(jnp.int32)
    x1 = x0 + 1
    y1 = y0 + 1

    base = (b * NV + start) * HEADS + hh

    def corner(xi, yi, wgt, c):
        vx = (xi >= 0) & (xi <= wi - 1)
        vy = (yi >= 0) & (yi <= wi - 1)
        xc = jnp.clip(xi, 0, wi - 1)
        yc = jnp.clip(yi, 0, wi - 1)
        rowidx = base + (yc * wi + xc) * HEADS
        wc = jnp.where(vx & vy, wgt * attn, 0.0)
        idx_ref[:, c * 128:(c + 1) * 128] = rowidx
        wgt_ref[:, c * 128:(c + 1) * 128] = wc

    corner(x0, y0, wx0 * wy0, 0)
    corner(x1, y0, wx1 * wy0, 1)
    corner(x0, y1, wx0 * wy1, 2)
    corner(x1, y1, wx1 * wy1, 3)


def _tc_sample_params(q_bf, w_oa_bf, b_oa, rp_wide):
    return pl.pallas_call(
        _sample_body,
        grid=(NBLK,),
        in_specs=[
            pl.BlockSpec((QBLK, EMBED), lambda i: (i, 0)),
            pl.BlockSpec((EMBED, 384), lambda i: (0, 0)),
            pl.BlockSpec((1, 384), lambda i: (0, 0)),
            pl.BlockSpec((QBLK, 256), lambda i: (i, 0)),
        ],
        out_specs=[
            pl.BlockSpec((QBLK, ENTRIES), lambda i: (i, 0)),
            pl.BlockSpec((QBLK, ENTRIES), lambda i: (i, 0)),
        ],
        out_shape=[
            jax.ShapeDtypeStruct((NQB, ENTRIES), jnp.int32),
            jax.ShapeDtypeStruct((NQB, ENTRIES), jnp.float32),
        ],
    )(q_bf, w_oa_bf, b_oa.reshape(1, 384), rp_wide)


_QCHUNK = 2
_ROWS = _QCHUNK * ENTRIES
_NWORK = 32
_PER_W = NQB // _NWORK


_NSTEP = _PER_W // _QCHUNK


def _sc_gather(vtab, idx, wgt):
    mesh = plsc.VectorSubcoreMesh(core_axis_name="c", subcore_axis_name="s")

    @functools.partial(
        pl.kernel,
        mesh=mesh,
        out_type=jax.ShapeDtypeStruct((BS * HEADS, NQ // 8, EMBED),
                                      jnp.float32),
        scratch_types=[
            pltpu.VMEM((2, _ROWS // 128, 128), jnp.int32),
            pltpu.VMEM((2, _ROWS), jnp.float32),
            pltpu.VMEM((2, _ROWS, HEAD_DIM), jnp.float32),
            pltpu.VMEM((HEADS, _QCHUNK * HEAD_DIM), jnp.float32),
            (pltpu.SemaphoreType.DMA, pltpu.SemaphoreType.DMA),
            (pltpu.SemaphoreType.DMA, pltpu.SemaphoreType.DMA),
        ],
        compiler_params=pltpu.CompilerParams(use_tc_tiling_on_sc=False),
    )
    def k(vtab_hbm, idx_hbm, w_hbm, out_hbm, idx_v, w_v, rows_v, out_v,
          sems, isems):
        wid = lax.axis_index("s") * 2 + lax.axis_index("c")
        base_q = wid * _PER_W

        def stage(step_i, buf):
            q0 = base_q + step_i * _QCHUNK
            e0 = q0 * ENTRIES
            pltpu.async_copy(idx_hbm.at[pl.ds(e0 // 128, _ROWS // 128)],
                             idx_v.at[buf], isems[buf])
            pltpu.async_copy(w_hbm.at[pl.ds(e0, _ROWS)], w_v.at[buf],
                             isems[buf])

        def stage_wait(step_i, buf):
            q0 = base_q + step_i * _QCHUNK
            e0 = q0 * ENTRIES
            pltpu.make_async_copy(
                idx_hbm.at[pl.ds(e0 // 128, _ROWS // 128)],
                idx_v.at[buf], isems[buf]).wait()
            pltpu.make_async_copy(
                w_hbm.at[pl.ds(e0, _ROWS)], w_v.at[buf], isems[buf]).wait()

        def fire(step_i, buf):
            for j in range(_ROWS // 128):
                pltpu.async_copy(vtab_hbm.at[idx_v.at[buf].at[j]],
                                 rows_v.at[buf].at[pl.ds(j * 128, 128)],
                                 sems[buf])

        def drain_compute(step_i, buf):
            for j in range(_ROWS // 128):
                pltpu.make_async_copy(
                    vtab_hbm.at[idx_v.at[buf].at[j]],
                    rows_v.at[buf].at[pl.ds(j * 128, 128)],
                    sems[buf]).wait()
            rbuf = rows_v.at[buf]
            for q in range(_QCHUNK):
                for h in range(HEADS):
                    def inner(c, accs):
                        aa = list(accs[0:4])
                        bb = list(accs[4:8])
                        kb = q * ENTRIES + c * 128 + h * 16
                        wvec = w_v[buf, pl.ds(kb, 16)]
                        for jj in range(16):
                            wv = wvec[jj]
                            aa[jj % 4] = aa[jj % 4] + wv * rbuf[kb + jj, 0:16]
                            bb[jj % 4] = bb[jj % 4] + wv * rbuf[kb + jj, 16:32]
                        return tuple(aa) + tuple(bb)

                    z = jnp.zeros((16,), jnp.float32)
                    accs = lax.fori_loop(0, 4, inner, (z,) * 8)
                    a0 = (accs[0] + accs[1]) + (accs[2] + accs[3])
                    a1 = (accs[4] + accs[5]) + (accs[6] + accs[7])
                    out_v[h, q * 32:q * 32 + 16] = a0
                    out_v[h, q * 32 + 16:q * 32 + 32] = a1
            q0 = base_q + step_i * _QCHUNK
            b8 = (q0 // NQ) * HEADS
            qm = q0 % NQ
            pltpu.sync_copy(
                out_v,
                out_hbm.at[pl.ds(b8, HEADS), qm // 8,
                           pl.ds((qm % 8) * HEAD_DIM,
                                 _QCHUNK * HEAD_DIM)])

        stage(0, 0)
        stage_wait(0, 0)
        fire(0, 0)
        stage(1, 1)

        def phase(g, buf):
            @pl.when(g + 1 < _NSTEP)
            def _():
                stage_wait(g + 1, buf ^ 1)
                fire(g + 1, buf ^ 1)

            drain_compute(g, buf)

            @pl.when(g + 2 < _NSTEP)
            def _():
                stage(g + 2, buf)

        def dbl(d, carry):
            g = d * 2
            phase(g, 0)
            phase(g + 1, 1)
            return carry

        lax.fori_loop(0, _NSTEP // 2, dbl, 0)

    return k(vtab, idx, wgt)


def kernel(query, reference_points, value, W_value, b_value, W_off, b_off,
           W_att, b_att, W_out, b_out, value_spatial_shapes,
           value_level_start_index):
    q2 = query.reshape(NQB, EMBED)
    v2 = value.reshape(NQB, EMBED)
    rp = reference_points.reshape(NQB, 2)
    rp_wide = jnp.concatenate(
        [jnp.broadcast_to(rp[:, 0:1], (NQB, 128)),
         jnp.broadcast_to(rp[:, 1:2], (NQB, 128))], axis=1)

    perm = jnp.asarray(_ATT_PERM, jnp.int32)
    w_oa = jnp.concatenate([W_off[:, 0::2], W_off[:, 1::2], W_att[:, perm]],
                           axis=1)
    b_oa = jnp.concatenate([b_off[0::2], b_off[1::2], b_att[perm]], axis=0)

    vtab = _tc_matmul(v2, W_value, b_value).reshape(NQB * HEADS, HEAD_DIM)
    idx, wgt = _tc_sample_params(q2, w_oa, b_oa, rp_wide)
    sc_out = _sc_gather(vtab, idx.reshape(-1, 128),
                        wgt.reshape(-1)).reshape(NQB, EMBED)
    out = _tc_matmul(sc_out, W_out, b_out)
    return out.reshape(BS, NQ, EMBED)
